# baseline (speedup 1.0000x reference)
import functools

import jax
import jax.numpy as jnp
from jax import lax
from jax.experimental import pallas as pl
from jax.experimental.pallas import tpu as pltpu
from jax.experimental.pallas import tpu_sc as plsc


def kernel(x, pos_table):
    B, T, D = x.shape
    NW = 32
    RPW = T // NW
    R = 8
    NSTEPS = RPW // R
    NB = 3

    x_flat = x.reshape(B * T, D)

    mesh = plsc.VectorSubcoreMesh(core_axis_name="c", subcore_axis_name="s")

    @functools.partial(
        pl.kernel,
        mesh=mesh,
        out_type=jax.ShapeDtypeStruct((B * T, D), jnp.float32),
        scratch_types=[
            pltpu.VMEM((NB, B, R, D), jnp.float32),
            pltpu.VMEM((NB, R, D), jnp.float32),
            pltpu.SemaphoreType.DMA((NB, B)),
            pltpu.SemaphoreType.DMA((NB,)),
            pltpu.SemaphoreType.DMA((NB, B)),
        ],
    )
    def sc_add(x_hbm, pos_hbm, out_hbm, x_buf, pos_buf, xsem, psem, osem):
        c = lax.axis_index("c")
        s = lax.axis_index("s")
        wid = c * 16 + s
        prow0 = wid * RPW

        def prow(k):
            return pl.multiple_of(prow0 + k * R, R)

        def xrow(k, b):
            return pl.multiple_of(b * T + prow0 + k * R, R)

        def start_loads(k):
            p = k % NB
            dxs = tuple(
                pltpu.async_copy(
                    x_hbm.at[pl.ds(xrow(k, b), R)], x_buf.at[p, b],
                    xsem.at[p, b])
                for b in range(B))
            dp = pltpu.async_copy(
                pos_hbm.at[pl.ds(prow(k), R)], pos_buf.at[p], psem.at[p])
            return dxs, dp

        loads = {0: start_loads(0)}
        stores = {}
        for k in range(NSTEPS):
            p = k % NB
            if k + 1 < NSTEPS:
                if k - 2 in stores:
                    for d in stores.pop(k - 2):
                        d.wait()
                loads[k + 1] = start_loads(k + 1)
            dxs, dp = loads.pop(k)
            for d in dxs:
                d.wait()
            dp.wait()

            @plsc.parallel_loop(0, B * R * D, step=16, unroll=8)
            def _(i):
                br = i // D
                b = br // R
                r = br % R
                d0 = pl.multiple_of(i % D, 16)
                sl = pl.ds(d0, 16)
                plsc.addupdate(x_buf.at[p, b, r].at[sl], pos_buf[p, r, sl])

            stores[k] = tuple(
                pltpu.async_copy(
                    x_buf.at[p, b], out_hbm.at[pl.ds(xrow(k, b), R)],
                    osem.at[p, b])
                for b in range(B))
        for k in sorted(stores):
            for d in stores.pop(k):
                d.wait()

    out = sc_add(x_flat, pos_table)
    return out.reshape(B, T, D)


# speedup vs baseline: 1.7735x; 1.7735x over previous
import jax
import jax.numpy as jnp
from jax.experimental import pallas as pl


def _add_body(x_ref, pos_ref, out_ref):
    out_ref[...] = x_ref[...] + pos_ref[...]


def kernel(x, pos_table):
    B, T, D = x.shape
    BT = 2048
    grid = (T // BT, B)
    return pl.pallas_call(
        _add_body,
        grid=grid,
        in_specs=[
            pl.BlockSpec((1, BT, D), lambda t, b: (b, t, 0)),
            pl.BlockSpec((BT, D), lambda t, b: (t, 0)),
        ],
        out_specs=pl.BlockSpec((1, BT, D), lambda t, b: (b, t, 0)),
        out_shape=jax.ShapeDtypeStruct((B, T, D), x.dtype),
    )(x, pos_table)
